# Initial kernel scaffold; baseline (speedup 1.0000x reference)
import jax
import jax.numpy as jnp
from jax.experimental import pallas as pl


def kernel(x, edge_index, lin1_w, lin1_b, lin2_w, lin2_b, filter_weights, wf_w, wf_b, wx_w, wx_b, lam):
    raise NotImplementedError("write your pallas kernel here")


# speedup vs baseline: 25.9038x; 25.9038x over previous
import functools

import jax
import jax.numpy as jnp
from jax.experimental import pallas as pl

N_NODES = 10000
HID = 128
F = 5

_ROW_BLK = 2000
_GRAM_BM = 512
_GRAM_BN = 512


def _front_kernel(x_ref, w1_ref, b1_ref, w2_ref, b2_ref, wf_ref, bf_ref,
                  wx_ref, bx_ref, lam_ref, res_ref, score_ref):
    x = x_ref[...]
    h = jnp.maximum(
        jax.lax.dot_general(x, w1_ref[...], (((1,), (1,)), ((), ())),
                            preferred_element_type=jnp.float32) + b1_ref[...],
        0.0)
    h = jax.lax.dot_general(h, w2_ref[...], (((1,), (1,)), ((), ())),
                            preferred_element_type=jnp.float32) + b2_ref[...]
    g = jax.lax.dot_general(h, wf_ref[...], (((1,), (1,)), ((), ())),
                            preferred_element_type=jnp.float32)
    xp = jnp.tanh(
        jax.lax.dot_general(h, wx_ref[...], (((1,), (1,)), ((), ())),
                            preferred_element_type=jnp.float32) + bx_ref[...])
    lams = jax.nn.softmax(lam_ref[...], axis=-1)
    logits = []
    for i in range(F):
        hp = jnp.tanh(lams[0, i] * g + bf_ref[...])
        logits.append(jnp.sum(hp * xp, axis=1, keepdims=True))
    logit = jnp.concatenate(logits, axis=1)
    m = jnp.max(logit, axis=1, keepdims=True)
    e = jnp.exp(logit - m)
    score = e / jnp.sum(e, axis=1, keepdims=True)
    s = jnp.sum(score * lams, axis=1, keepdims=True)
    res_ref[...] = h * s
    score_ref[...] = score


def _gram_kernel(a_ref, b_ref, o_ref):
    o_ref[...] = jax.lax.dot_general(
        a_ref[...], b_ref[...], (((1,), (1,)), ((), ())),
        preferred_element_type=jnp.float32)


@functools.partial(jax.jit, static_argnames=())
def kernel(x, edge_index, lin1_w, lin1_b, lin2_w, lin2_b, filter_weights,
           wf_w, wf_b, wx_w, wx_b, lam):
    n = x.shape[0]
    b1 = lin1_b.reshape(1, HID)
    b2 = lin2_b.reshape(1, HID)
    bf = wf_b.reshape(1, HID)
    bx = wx_b.reshape(1, HID)
    lam2 = lam.reshape(1, F)

    grid_a = n // _ROW_BLK
    full = lambda shp: pl.BlockSpec(shp, lambda i: (0, 0))
    res, score = pl.pallas_call(
        _front_kernel,
        grid=(grid_a,),
        in_specs=[
            pl.BlockSpec((_ROW_BLK, HID), lambda i: (i, 0)),
            full((HID, HID)), full((1, HID)),
            full((HID, HID)), full((1, HID)),
            full((HID, HID)), full((1, HID)),
            full((HID, HID)), full((1, HID)),
            full((1, F)),
        ],
        out_specs=[
            pl.BlockSpec((_ROW_BLK, HID), lambda i: (i, 0)),
            pl.BlockSpec((_ROW_BLK, F), lambda i: (i, 0)),
        ],
        out_shape=[
            jax.ShapeDtypeStruct((n, HID), jnp.float32),
            jax.ShapeDtypeStruct((n, F), jnp.float32),
        ],
    )(x, lin1_w, b1, lin2_w, b2, wf_w, bf, wx_w, bx, lam2)

    gm = pl.cdiv(n, _GRAM_BM)
    gn = pl.cdiv(n, _GRAM_BN)
    res_ = pl.pallas_call(
        _gram_kernel,
        grid=(gm, gn),
        in_specs=[
            pl.BlockSpec((_GRAM_BM, HID), lambda i, j: (i, 0)),
            pl.BlockSpec((_GRAM_BN, HID), lambda i, j: (j, 0)),
        ],
        out_specs=pl.BlockSpec((_GRAM_BM, _GRAM_BN), lambda i, j: (i, j)),
        out_shape=jax.ShapeDtypeStruct((n, n), jnp.float32),
    )(res, res)

    return (res_, res, score.T)


# speedup vs baseline: 25.9105x; 1.0003x over previous
import functools

import jax
import jax.numpy as jnp
from jax.experimental import pallas as pl

N_NODES = 10000
HID = 128
F = 5

_ROW_BLK = 2000
_GRAM_BM = 512
_GRAM_BN = 512


def _front_kernel(x_ref, w1_ref, b1_ref, w2_ref, b2_ref, wf_ref, bf_ref,
                  wx_ref, bx_ref, lam_ref, res_ref, score_ref):
    x = x_ref[...]
    h = jnp.maximum(
        jax.lax.dot_general(x, w1_ref[...], (((1,), (1,)), ((), ())),
                            preferred_element_type=jnp.float32) + b1_ref[...],
        0.0)
    h = jax.lax.dot_general(h, w2_ref[...], (((1,), (1,)), ((), ())),
                            preferred_element_type=jnp.float32) + b2_ref[...]
    g = jax.lax.dot_general(h, wf_ref[...], (((1,), (1,)), ((), ())),
                            preferred_element_type=jnp.float32)
    xp = jnp.tanh(
        jax.lax.dot_general(h, wx_ref[...], (((1,), (1,)), ((), ())),
                            preferred_element_type=jnp.float32) + bx_ref[...])
    lams = jax.nn.softmax(lam_ref[...], axis=-1)
    logits = []
    for i in range(F):
        hp = jnp.tanh(lams[0, i] * g + bf_ref[...])
        logits.append(jnp.sum(hp * xp, axis=1, keepdims=True))
    logit = jnp.concatenate(logits, axis=1)
    m = jnp.max(logit, axis=1, keepdims=True)
    e = jnp.exp(logit - m)
    score = e / jnp.sum(e, axis=1, keepdims=True)
    s = jnp.sum(score * lams, axis=1, keepdims=True)
    res_ref[...] = h * s
    score_ref[...] = score


def _gram_kernel(a_ref, b_ref, o_ref):
    o_ref[...] = jax.lax.dot_general(
        a_ref[...].astype(jnp.bfloat16), b_ref[...].astype(jnp.bfloat16),
        (((1,), (1,)), ((), ())),
        preferred_element_type=jnp.float32)


@functools.partial(jax.jit, static_argnames=())
def kernel(x, edge_index, lin1_w, lin1_b, lin2_w, lin2_b, filter_weights,
           wf_w, wf_b, wx_w, wx_b, lam):
    n = x.shape[0]
    b1 = lin1_b.reshape(1, HID)
    b2 = lin2_b.reshape(1, HID)
    bf = wf_b.reshape(1, HID)
    bx = wx_b.reshape(1, HID)
    lam2 = lam.reshape(1, F)

    grid_a = n // _ROW_BLK
    full = lambda shp: pl.BlockSpec(shp, lambda i: (0, 0))
    res, score = pl.pallas_call(
        _front_kernel,
        grid=(grid_a,),
        in_specs=[
            pl.BlockSpec((_ROW_BLK, HID), lambda i: (i, 0)),
            full((HID, HID)), full((1, HID)),
            full((HID, HID)), full((1, HID)),
            full((HID, HID)), full((1, HID)),
            full((HID, HID)), full((1, HID)),
            full((1, F)),
        ],
        out_specs=[
            pl.BlockSpec((_ROW_BLK, HID), lambda i: (i, 0)),
            pl.BlockSpec((_ROW_BLK, F), lambda i: (i, 0)),
        ],
        out_shape=[
            jax.ShapeDtypeStruct((n, HID), jnp.float32),
            jax.ShapeDtypeStruct((n, F), jnp.float32),
        ],
    )(x, lin1_w, b1, lin2_w, b2, wf_w, bf, wx_w, bx, lam2)

    gm = pl.cdiv(n, _GRAM_BM)
    gn = pl.cdiv(n, _GRAM_BN)
    res_ = pl.pallas_call(
        _gram_kernel,
        grid=(gm, gn),
        in_specs=[
            pl.BlockSpec((_GRAM_BM, HID), lambda i, j: (i, 0)),
            pl.BlockSpec((_GRAM_BN, HID), lambda i, j: (j, 0)),
        ],
        out_specs=pl.BlockSpec((_GRAM_BM, _GRAM_BN), lambda i, j: (i, j)),
        out_shape=jax.ShapeDtypeStruct((n, n), jnp.float32),
    )(res, res)

    return (res_, res, score.T)


# speedup vs baseline: 65.6048x; 2.5320x over previous
import functools

import jax
import jax.numpy as jnp
from jax.experimental import pallas as pl

N_NODES = 10000
HID = 128
F = 5

_ROW_BLK = 2000
_GRAM_BM = 400


def _front_kernel(x_ref, w1_ref, b1_ref, w2_ref, b2_ref, wf_ref, bf_ref,
                  wx_ref, bx_ref, lam_ref, res_ref, score_ref):
    x = x_ref[...]
    h = jnp.maximum(
        jax.lax.dot_general(x, w1_ref[...], (((1,), (1,)), ((), ())),
                            preferred_element_type=jnp.float32) + b1_ref[...],
        0.0)
    h = jax.lax.dot_general(h, w2_ref[...], (((1,), (1,)), ((), ())),
                            preferred_element_type=jnp.float32) + b2_ref[...]
    g = jax.lax.dot_general(h, wf_ref[...], (((1,), (1,)), ((), ())),
                            preferred_element_type=jnp.float32)
    xp = jnp.tanh(
        jax.lax.dot_general(h, wx_ref[...], (((1,), (1,)), ((), ())),
                            preferred_element_type=jnp.float32) + bx_ref[...])
    lams = jax.nn.softmax(lam_ref[...], axis=-1)
    logits = []
    for i in range(F):
        hp = jnp.tanh(lams[0, i] * g + bf_ref[...])
        logits.append(jnp.sum(hp * xp, axis=1, keepdims=True))
    logit = jnp.concatenate(logits, axis=1)
    m = jnp.max(logit, axis=1, keepdims=True)
    e = jnp.exp(logit - m)
    score = e / jnp.sum(e, axis=1, keepdims=True)
    s = jnp.sum(score * lams, axis=1, keepdims=True)
    res_ref[...] = h * s
    score_ref[...] = score


def _gram_kernel(a_ref, b_ref, o_ref):
    o_ref[...] = jax.lax.dot_general(
        a_ref[...].astype(jnp.bfloat16), b_ref[...].astype(jnp.bfloat16),
        (((1,), (1,)), ((), ())),
        preferred_element_type=jnp.float32)


@functools.partial(jax.jit, static_argnames=())
def kernel(x, edge_index, lin1_w, lin1_b, lin2_w, lin2_b, filter_weights,
           wf_w, wf_b, wx_w, wx_b, lam):
    n = x.shape[0]
    b1 = lin1_b.reshape(1, HID)
    b2 = lin2_b.reshape(1, HID)
    bf = wf_b.reshape(1, HID)
    bx = wx_b.reshape(1, HID)
    lam2 = lam.reshape(1, F)

    grid_a = n // _ROW_BLK
    full = lambda shp: pl.BlockSpec(shp, lambda i: (0, 0))
    res, score = pl.pallas_call(
        _front_kernel,
        grid=(grid_a,),
        in_specs=[
            pl.BlockSpec((_ROW_BLK, HID), lambda i: (i, 0)),
            full((HID, HID)), full((1, HID)),
            full((HID, HID)), full((1, HID)),
            full((HID, HID)), full((1, HID)),
            full((HID, HID)), full((1, HID)),
            full((1, F)),
        ],
        out_specs=[
            pl.BlockSpec((_ROW_BLK, HID), lambda i: (i, 0)),
            pl.BlockSpec((_ROW_BLK, F), lambda i: (i, 0)),
        ],
        out_shape=[
            jax.ShapeDtypeStruct((n, HID), jnp.float32),
            jax.ShapeDtypeStruct((n, F), jnp.float32),
        ],
    )(x, lin1_w, b1, lin2_w, b2, wf_w, bf, wx_w, bx, lam2)

    gm = pl.cdiv(n, _GRAM_BM)
    res_ = pl.pallas_call(
        _gram_kernel,
        grid=(gm,),
        in_specs=[
            pl.BlockSpec((_GRAM_BM, HID), lambda i: (i, 0)),
            pl.BlockSpec((n, HID), lambda i: (0, 0)),
        ],
        out_specs=pl.BlockSpec((_GRAM_BM, n), lambda i: (i, 0)),
        out_shape=jax.ShapeDtypeStruct((n, n), jnp.float32),
    )(res, res)

    return (res_, res, score.T)


# speedup vs baseline: 65.9134x; 1.0047x over previous
import functools

import jax
import jax.numpy as jnp
from jax.experimental import pallas as pl

N_NODES = 10000
HID = 128
F = 5

_ROW_BLK = 2000
_GRAM_BM = 400


def _front_kernel(x_ref, w1_ref, b1_ref, w2_ref, b2_ref, wf_ref, bf_ref,
                  wx_ref, bx_ref, lam_ref, res_ref, resb_ref, score_ref):
    x = x_ref[...]
    h = jnp.maximum(
        jax.lax.dot_general(x, w1_ref[...], (((1,), (1,)), ((), ())),
                            preferred_element_type=jnp.float32) + b1_ref[...],
        0.0)
    h = jax.lax.dot_general(h, w2_ref[...], (((1,), (1,)), ((), ())),
                            preferred_element_type=jnp.float32) + b2_ref[...]
    g = jax.lax.dot_general(h, wf_ref[...], (((1,), (1,)), ((), ())),
                            preferred_element_type=jnp.float32)
    xp = jnp.tanh(
        jax.lax.dot_general(h, wx_ref[...], (((1,), (1,)), ((), ())),
                            preferred_element_type=jnp.float32) + bx_ref[...])
    lams = jax.nn.softmax(lam_ref[...], axis=-1)
    logits = []
    for i in range(F):
        hp = jnp.tanh(lams[0, i] * g + bf_ref[...])
        logits.append(jnp.sum(hp * xp, axis=1, keepdims=True))
    logit = jnp.concatenate(logits, axis=1)
    m = jnp.max(logit, axis=1, keepdims=True)
    e = jnp.exp(logit - m)
    score = e / jnp.sum(e, axis=1, keepdims=True)
    s = jnp.sum(score * lams, axis=1, keepdims=True)
    r = h * s
    res_ref[...] = r
    resb_ref[...] = r.astype(jnp.bfloat16)
    score_ref[...] = score


def _gram_kernel(a_ref, b_ref, o_ref):
    o_ref[...] = jax.lax.dot_general(
        a_ref[...], b_ref[...], (((1,), (1,)), ((), ())),
        preferred_element_type=jnp.float32)


@functools.partial(jax.jit, static_argnames=())
def kernel(x, edge_index, lin1_w, lin1_b, lin2_w, lin2_b, filter_weights,
           wf_w, wf_b, wx_w, wx_b, lam):
    n = x.shape[0]
    b1 = lin1_b.reshape(1, HID)
    b2 = lin2_b.reshape(1, HID)
    bf = wf_b.reshape(1, HID)
    bx = wx_b.reshape(1, HID)
    lam2 = lam.reshape(1, F)

    grid_a = n // _ROW_BLK
    full = lambda shp: pl.BlockSpec(shp, lambda i: (0, 0))
    res, res_b16, score = pl.pallas_call(
        _front_kernel,
        grid=(grid_a,),
        in_specs=[
            pl.BlockSpec((_ROW_BLK, HID), lambda i: (i, 0)),
            full((HID, HID)), full((1, HID)),
            full((HID, HID)), full((1, HID)),
            full((HID, HID)), full((1, HID)),
            full((HID, HID)), full((1, HID)),
            full((1, F)),
        ],
        out_specs=[
            pl.BlockSpec((_ROW_BLK, HID), lambda i: (i, 0)),
            pl.BlockSpec((_ROW_BLK, HID), lambda i: (i, 0)),
            pl.BlockSpec((_ROW_BLK, F), lambda i: (i, 0)),
        ],
        out_shape=[
            jax.ShapeDtypeStruct((n, HID), jnp.float32),
            jax.ShapeDtypeStruct((n, HID), jnp.bfloat16),
            jax.ShapeDtypeStruct((n, F), jnp.float32),
        ],
    )(x, lin1_w, b1, lin2_w, b2, wf_w, bf, wx_w, bx, lam2)

    gm = pl.cdiv(n, _GRAM_BM)
    res_ = pl.pallas_call(
        _gram_kernel,
        grid=(gm,),
        in_specs=[
            pl.BlockSpec((_GRAM_BM, HID), lambda i: (i, 0)),
            pl.BlockSpec((n, HID), lambda i: (0, 0)),
        ],
        out_specs=pl.BlockSpec((_GRAM_BM, n), lambda i: (i, 0)),
        out_shape=jax.ShapeDtypeStruct((n, n), jnp.float32),
    )(res_b16, res_b16)

    return (res_, res, score.T)


# speedup vs baseline: 66.0900x; 1.0027x over previous
import functools

import jax
import jax.numpy as jnp
from jax.experimental import pallas as pl

N_NODES = 10000
HID = 128
F = 5

_ROW_BLK = 2000
_GRAM_BM = 200


def _front_kernel(x_ref, w1_ref, b1_ref, w2_ref, b2_ref, wf_ref, bf_ref,
                  wx_ref, bx_ref, lam_ref, res_ref, resb_ref, score_ref):
    x = x_ref[...]
    h = jnp.maximum(
        jax.lax.dot_general(x, w1_ref[...], (((1,), (1,)), ((), ())),
                            preferred_element_type=jnp.float32) + b1_ref[...],
        0.0)
    h = jax.lax.dot_general(h, w2_ref[...], (((1,), (1,)), ((), ())),
                            preferred_element_type=jnp.float32) + b2_ref[...]
    g = jax.lax.dot_general(h, wf_ref[...], (((1,), (1,)), ((), ())),
                            preferred_element_type=jnp.float32)
    xp = jnp.tanh(
        jax.lax.dot_general(h, wx_ref[...], (((1,), (1,)), ((), ())),
                            preferred_element_type=jnp.float32) + bx_ref[...])
    lams = jax.nn.softmax(lam_ref[...], axis=-1)
    logits = []
    for i in range(F):
        hp = jnp.tanh(lams[0, i] * g + bf_ref[...])
        logits.append(jnp.sum(hp * xp, axis=1, keepdims=True))
    logit = jnp.concatenate(logits, axis=1)
    m = jnp.max(logit, axis=1, keepdims=True)
    e = jnp.exp(logit - m)
    score = e / jnp.sum(e, axis=1, keepdims=True)
    s = jnp.sum(score * lams, axis=1, keepdims=True)
    r = h * s
    res_ref[...] = r
    resb_ref[...] = r.astype(jnp.bfloat16)
    score_ref[...] = score


def _gram_kernel(a_ref, b_ref, o_ref):
    o_ref[...] = jax.lax.dot_general(
        a_ref[...], b_ref[...], (((1,), (1,)), ((), ())),
        preferred_element_type=jnp.float32)


@functools.partial(jax.jit, static_argnames=())
def kernel(x, edge_index, lin1_w, lin1_b, lin2_w, lin2_b, filter_weights,
           wf_w, wf_b, wx_w, wx_b, lam):
    n = x.shape[0]
    b1 = lin1_b.reshape(1, HID)
    b2 = lin2_b.reshape(1, HID)
    bf = wf_b.reshape(1, HID)
    bx = wx_b.reshape(1, HID)
    lam2 = lam.reshape(1, F)

    grid_a = n // _ROW_BLK
    full = lambda shp: pl.BlockSpec(shp, lambda i: (0, 0))
    res, res_b16, score = pl.pallas_call(
        _front_kernel,
        grid=(grid_a,),
        in_specs=[
            pl.BlockSpec((_ROW_BLK, HID), lambda i: (i, 0)),
            full((HID, HID)), full((1, HID)),
            full((HID, HID)), full((1, HID)),
            full((HID, HID)), full((1, HID)),
            full((HID, HID)), full((1, HID)),
            full((1, F)),
        ],
        out_specs=[
            pl.BlockSpec((_ROW_BLK, HID), lambda i: (i, 0)),
            pl.BlockSpec((_ROW_BLK, HID), lambda i: (i, 0)),
            pl.BlockSpec((_ROW_BLK, F), lambda i: (i, 0)),
        ],
        out_shape=[
            jax.ShapeDtypeStruct((n, HID), jnp.float32),
            jax.ShapeDtypeStruct((n, HID), jnp.bfloat16),
            jax.ShapeDtypeStruct((n, F), jnp.float32),
        ],
    )(x, lin1_w, b1, lin2_w, b2, wf_w, bf, wx_w, bx, lam2)

    gm = pl.cdiv(n, _GRAM_BM)
    res_ = pl.pallas_call(
        _gram_kernel,
        grid=(gm,),
        in_specs=[
            pl.BlockSpec((_GRAM_BM, HID), lambda i: (i, 0)),
            pl.BlockSpec((n, HID), lambda i: (0, 0)),
        ],
        out_specs=pl.BlockSpec((_GRAM_BM, n), lambda i: (i, 0)),
        out_shape=jax.ShapeDtypeStruct((n, n), jnp.float32),
    )(res_b16, res_b16)

    return (res_, res, score.T)


# speedup vs baseline: 67.7423x; 1.0250x over previous
import functools

import jax
import jax.numpy as jnp
from jax.experimental import pallas as pl
from jax.experimental.pallas import tpu as pltpu

HID = 128
F = 5

_BM = 200


def _fused_kernel(x_ref, w1_ref, b1_ref, w2_ref, b2_ref, wf_ref, bf_ref,
                  wx_ref, bx_ref, lam_ref, gram_ref, res_ref, score_ref,
                  resb_ref):
    i = pl.program_id(0)

    @pl.when(i == 0)
    def _front():
        x = x_ref[...]
        h = jnp.maximum(
            jax.lax.dot_general(x, w1_ref[...], (((1,), (1,)), ((), ())),
                                preferred_element_type=jnp.float32)
            + b1_ref[...], 0.0)
        h = jax.lax.dot_general(h, w2_ref[...], (((1,), (1,)), ((), ())),
                                preferred_element_type=jnp.float32) + b2_ref[...]
        g = jax.lax.dot_general(h, wf_ref[...], (((1,), (1,)), ((), ())),
                                preferred_element_type=jnp.float32)
        xp = jnp.tanh(
            jax.lax.dot_general(h, wx_ref[...], (((1,), (1,)), ((), ())),
                                preferred_element_type=jnp.float32)
            + bx_ref[...])
        lams = jax.nn.softmax(lam_ref[...], axis=-1)
        logits = []
        for k in range(F):
            hp = jnp.tanh(lams[0, k] * g + bf_ref[...])
            logits.append(jnp.sum(hp * xp, axis=1, keepdims=True))
        logit = jnp.concatenate(logits, axis=1)
        m = jnp.max(logit, axis=1, keepdims=True)
        e = jnp.exp(logit - m)
        score = e / jnp.sum(e, axis=1, keepdims=True)
        s = jnp.sum(score * lams, axis=1, keepdims=True)
        r = h * s
        res_ref[...] = r
        score_ref[...] = score
        resb_ref[...] = r.astype(jnp.bfloat16)

    a = resb_ref[pl.ds(i * _BM, _BM), :]
    gram_ref[...] = jax.lax.dot_general(
        a, resb_ref[...], (((1,), (1,)), ((), ())),
        preferred_element_type=jnp.float32)


@functools.partial(jax.jit, static_argnames=())
def kernel(x, edge_index, lin1_w, lin1_b, lin2_w, lin2_b, filter_weights,
           wf_w, wf_b, wx_w, wx_b, lam):
    n = x.shape[0]
    b1 = lin1_b.reshape(1, HID)
    b2 = lin2_b.reshape(1, HID)
    bf = wf_b.reshape(1, HID)
    bx = wx_b.reshape(1, HID)
    lam2 = lam.reshape(1, F)

    full = lambda shp: pl.BlockSpec(shp, lambda i: (0, 0))
    res_, res, score = pl.pallas_call(
        _fused_kernel,
        grid=(n // _BM,),
        in_specs=[
            full((n, HID)),
            full((HID, HID)), full((1, HID)),
            full((HID, HID)), full((1, HID)),
            full((HID, HID)), full((1, HID)),
            full((HID, HID)), full((1, HID)),
            full((1, F)),
        ],
        out_specs=[
            pl.BlockSpec((_BM, n), lambda i: (i, 0)),
            full((n, HID)),
            full((n, F)),
        ],
        out_shape=[
            jax.ShapeDtypeStruct((n, n), jnp.float32),
            jax.ShapeDtypeStruct((n, HID), jnp.float32),
            jax.ShapeDtypeStruct((n, F), jnp.float32),
        ],
        scratch_shapes=[pltpu.VMEM((n, HID), jnp.bfloat16)],
    )(x, lin1_w, b1, lin2_w, b2, wf_w, bf, wx_w, bx, lam2)

    return (res_, res, score.T)
